# baseline (speedup 1.0000x reference)
import functools

import jax
import jax.numpy as jnp
from jax import lax
from jax.experimental import pallas as pl
from jax.experimental.pallas import tpu as pltpu
from jax.experimental.pallas import tpu_sc as plsc

N = 10000
D = 128
E = 320000


NSUB = 16
CH = 128
NCHT = E // CH
IB = 160
NPAD = NSUB * IB - NCHT
NA = N + 48
WB = 640
NB = 2
IBLK = 32
NBLK = IB // IBLK

@functools.cache
def _seg2_built():
    mesh = plsc.VectorSubcoreMesh(core_axis_name="c", subcore_axis_name="s")
    return functools.partial(
        pl.kernel,
        mesh=mesh,
        out_type=(
            jax.ShapeDtypeStruct((N, D), jnp.float32),
            jax.ShapeDtypeStruct((N, D), jnp.float32),
        ),
        scratch_types=[
            pltpu.VMEM((IBLK, CH), jnp.int32),
            pltpu.VMEM((IBLK, CH), jnp.int32),
        ]
        + [pltpu.VMEM((CH, D), jnp.float32) for _ in range(NB)]
        + [
            pltpu.VMEM_SHARED((NA, D), jnp.float32),
        ]
        + [pltpu.SemaphoreType.DMA for _ in range(NB)],
    )(_seg2_body)


NCH2 = NSUB * IB


def _pad_body(ea_ref, eb_ref, sa_ref, da_ref, sb_ref, db_ref):
    zs = jnp.zeros((NPAD, CH), jnp.int32)
    zd = N + lax.broadcasted_iota(jnp.int32, (NPAD, CH), 1) % 32

    def pack(rows, pad):
        t = jnp.concatenate([rows, pad], axis=0)
        return t.reshape(IB, NSUB, CH).transpose(1, 0, 2).reshape(NCH2, CH)

    sa_ref[...] = pack(ea_ref[0], zs)
    da_ref[...] = pack(ea_ref[1], zd)
    sb_ref[...] = pack(eb_ref[0], zs)
    db_ref[...] = pack(eb_ref[1], zd)


def _pad_tables(ea, eb):
    shp = jax.ShapeDtypeStruct((NCH2, CH), jnp.int32)
    return pl.pallas_call(
        _pad_body,
        out_shape=[shp] * 4,
    )(ea.reshape(2, NCHT, CH), eb.reshape(2, NCHT, CH))


def _seg2(ha, hb, tables):
    sa, da, sb, db = tables
    return _seg2_built()(ha, hb, sa, da, sb, db)


def _seg2_body(ha, hb, ea_s, ea_d, eb_s, eb_d, oa, ob,
               isrc, idst, *rest):
    rows = rest[:NB]
    accum = rest[NB]
    gsems = rest[NB + 1:]
    c = lax.axis_index("c")
    s = lax.axis_index("s")

    zv = jnp.zeros((16,), jnp.float32)

    def zrow(i, carry):
        for j in range(D // 16):
            rows[0][i, pl.ds(j * 16, 16)] = zv
        return carry

    lax.fori_loop(0, CH, zrow, 0)
    base_r = jnp.minimum(s * WB, N - WB)
    for k in range(WB // CH):
        pltpu.sync_copy(rows[0], accum.at[pl.ds(base_r + k * CH, CH)])
    plsc.subcore_barrier()

    cstart = IB * s

    def run(h_ref, es_ref, ed_ref):
        def blk(kblk, carry):
            b0 = cstart + kblk * IBLK
            pltpu.sync_copy(es_ref.at[pl.ds(b0, IBLK)], isrc)
            pltpu.sync_copy(ed_ref.at[pl.ds(b0, IBLK)], idst)

            def chunk(i, carry):
                pltpu.sync_copy(h_ref.at[isrc.at[i]], rows[0])
                pltpu.sync_copy(rows[0], accum.at[idst.at[i]], add=True)
                return carry

            lax.fori_loop(0, IBLK, chunk, 0)
            return carry

        lax.fori_loop(0, NBLK, blk, 0)

    @pl.when(c == 0)
    def _():
        run(ha, ea_s, ea_d)

    @pl.when(c == 1)
    def _():
        run(hb, eb_s, eb_d)

    plsc.subcore_barrier()

    @pl.when(c == 0)
    def _():
        pltpu.sync_copy(accum.at[pl.ds(base_r, WB)], oa.at[pl.ds(base_r, WB)])

    @pl.when(c == 1)
    def _():
        pltpu.sync_copy(accum.at[pl.ds(base_r, WB)], ob.at[pl.ds(base_r, WB)])


RB = 1000
GRID = N // RB

_row_spec = pl.BlockSpec((RB, D), lambda r: (r, 0))
_w_spec = pl.BlockSpec((D, D), lambda r: (0, 0))
_b_spec = pl.BlockSpec((1, D), lambda r: (0, 0))
_row_shape = jax.ShapeDtypeStruct((N, D), jnp.float32)

_INV_SQRT2 = 0.7071067811865476


def _gelu(t):
    return 0.5 * t * (1.0 + lax.erf(t * _INV_SQRT2))


def _mm2_body(x_ref, wa_ref, wb_ref, oa_ref, ob_ref):
    xb = x_ref[...]
    oa_ref[...] = jnp.dot(xb, wa_ref[...], preferred_element_type=jnp.float32)
    ob_ref[...] = jnp.dot(xb, wb_ref[...], preferred_element_type=jnp.float32)


def _mm2(x, wa, wb):
    return pl.pallas_call(
        _mm2_body,
        grid=(GRID,),
        in_specs=[_row_spec, _w_spec, _w_spec],
        out_specs=[_row_spec, _row_spec],
        out_shape=[_row_shape, _row_shape],
    )(x, wa, wb)


def _comb_body(aa_ref, ab_ref, ha_ref, hb_ref, ba_ref, bb_ref,
               wa_ref, wb_ref, oa_ref, ob_ref):
    t = (aa_ref[...] + ab_ref[...] + ha_ref[...] + hb_ref[...]
         + ba_ref[...] + bb_ref[...])
    h = _gelu(t)
    oa_ref[...] = jnp.dot(h, wa_ref[...], preferred_element_type=jnp.float32)
    ob_ref[...] = jnp.dot(h, wb_ref[...], preferred_element_type=jnp.float32)


def _comb_mm2(aa, ab, ha, hb, ba, bb, wa, wb):
    return pl.pallas_call(
        _comb_body,
        grid=(GRID,),
        in_specs=[_row_spec, _row_spec, _row_spec, _row_spec,
                  _b_spec, _b_spec, _w_spec, _w_spec],
        out_specs=[_row_spec, _row_spec],
        out_shape=[_row_shape, _row_shape],
    )(aa, ab, ha, hb, ba, bb, wa, wb)


def _head_body(aa_ref, ab_ref, ha_ref, hb_ref, ba_ref, bb_ref,
               w1_ref, b1_ref, w2_ref, b2_ref, o_ref):
    t = (aa_ref[...] + ab_ref[...] + ha_ref[...] + hb_ref[...]
         + ba_ref[...] + bb_ref[...])
    h = _gelu(t)
    h = _gelu(jnp.dot(h, w1_ref[...], preferred_element_type=jnp.float32)
              + b1_ref[...])
    o_ref[...] = (jnp.dot(h, w2_ref[...], preferred_element_type=jnp.float32)
                  + b2_ref[...])


def _head(aa, ab, ha, hb, ba, bb, w1, b1, w2, b2):
    return pl.pallas_call(
        _head_body,
        grid=(GRID,),
        in_specs=[_row_spec, _row_spec, _row_spec, _row_spec,
                  _b_spec, _b_spec, _w_spec, _b_spec, _w_spec, _b_spec],
        out_specs=_row_spec,
        out_shape=_row_shape,
    )(aa, ab, ha, hb, ba, bb, w1, b1, w2, b2)


def kernel(x, edge_index_a, edge_index_b,
           W0a, b0a, W0b, b0b, W1a, b1a, W1b, b1b,
           Wh1, bh1, Wh2, bh2):
    tables = _pad_tables(edge_index_a, edge_index_b)
    ha, hb = _mm2(x, W0a, W0b)
    aa, ab = _seg2(ha, hb, tables)
    h1a, h1b = _comb_mm2(aa, ab, ha, hb,
                         b0a.reshape(1, D), b0b.reshape(1, D), W1a, W1b)
    a1a, a1b = _seg2(h1a, h1b, tables)
    out = _head(a1a, a1b, h1a, h1b,
                b1a.reshape(1, D), b1b.reshape(1, D),
                Wh1, bh1.reshape(1, D), Wh2, bh2.reshape(1, D))
    return out


# speedup vs baseline: 1.5029x; 1.5029x over previous
import functools

import jax
import jax.numpy as jnp
from jax import lax
from jax.experimental import pallas as pl
from jax.experimental.pallas import tpu as pltpu
from jax.experimental.pallas import tpu_sc as plsc

N = 10000
D = 128
E = 320000


NSUB = 16
CH = 128
NCHT = E // CH
WB = 640


@functools.cache
def _seg2_built():
    mesh = plsc.VectorSubcoreMesh(core_axis_name="c", subcore_axis_name="s")
    return functools.partial(
        pl.kernel,
        mesh=mesh,
        out_type=(
            jax.ShapeDtypeStruct((N, D), jnp.float32),
            jax.ShapeDtypeStruct((N, D), jnp.float32),
        ),
        scratch_types=[
            pltpu.VMEM((2, CH), jnp.int32),
            pltpu.VMEM((CH, D), jnp.float32),
            pltpu.VMEM_SHARED((N, D), jnp.float32),
        ],
    )(_seg2_body)


def _seg2(ha, hb, ea, eb):
    return _seg2_built()(ha, hb, ea, eb)


def _seg2_body(ha, hb, ea, eb, oa, ob, idx, rows, accum):
    c = lax.axis_index("c")
    s = lax.axis_index("s")

    zv = jnp.zeros((16,), jnp.float32)

    def zrow(i, carry):
        for j in range(D // 16):
            rows[i, pl.ds(j * 16, 16)] = zv
        return carry

    lax.fori_loop(0, CH, zrow, 0)
    base_r = jnp.minimum(s * WB, N - WB)
    for k in range(WB // CH):
        pltpu.sync_copy(rows, accum.at[pl.ds(base_r + k * CH, CH)])
    plsc.subcore_barrier()

    cnt = jnp.where(s < NCHT % NSUB, NCHT // NSUB + 1, NCHT // NSUB)

    def run(h_ref, e_ref):
        def chunk(k, carry):
            j = s + NSUB * k
            pltpu.sync_copy(e_ref.at[:, pl.ds(j * CH, CH)], idx)
            pltpu.sync_copy(h_ref.at[idx.at[0]], rows)
            pltpu.sync_copy(rows, accum.at[idx.at[1]], add=True)
            return carry

        lax.fori_loop(0, cnt, chunk, 0)

    @pl.when(c == 0)
    def _():
        run(ha, ea)

    @pl.when(c == 1)
    def _():
        run(hb, eb)

    plsc.subcore_barrier()

    @pl.when(c == 0)
    def _():
        pltpu.sync_copy(accum.at[pl.ds(base_r, WB)], oa.at[pl.ds(base_r, WB)])

    @pl.when(c == 1)
    def _():
        pltpu.sync_copy(accum.at[pl.ds(base_r, WB)], ob.at[pl.ds(base_r, WB)])


RB = 1000
GRID = N // RB

_row_spec = pl.BlockSpec((RB, D), lambda r: (r, 0))
_w_spec = pl.BlockSpec((D, D), lambda r: (0, 0))
_b_spec = pl.BlockSpec((1, D), lambda r: (0, 0))
_row_shape = jax.ShapeDtypeStruct((N, D), jnp.float32)

_INV_SQRT2 = 0.7071067811865476


def _gelu(t):
    return 0.5 * t * (1.0 + lax.erf(t * _INV_SQRT2))


def _mm2_body(x_ref, wa_ref, wb_ref, oa_ref, ob_ref):
    xb = x_ref[...]
    oa_ref[...] = jnp.dot(xb, wa_ref[...], preferred_element_type=jnp.float32)
    ob_ref[...] = jnp.dot(xb, wb_ref[...], preferred_element_type=jnp.float32)


def _mm2(x, wa, wb):
    return pl.pallas_call(
        _mm2_body,
        grid=(GRID,),
        in_specs=[_row_spec, _w_spec, _w_spec],
        out_specs=[_row_spec, _row_spec],
        out_shape=[_row_shape, _row_shape],
    )(x, wa, wb)


def _comb_body(aa_ref, ab_ref, ha_ref, hb_ref, ba_ref, bb_ref,
               wa_ref, wb_ref, oa_ref, ob_ref):
    t = (aa_ref[...] + ab_ref[...] + ha_ref[...] + hb_ref[...]
         + ba_ref[...] + bb_ref[...])
    h = _gelu(t)
    oa_ref[...] = jnp.dot(h, wa_ref[...], preferred_element_type=jnp.float32)
    ob_ref[...] = jnp.dot(h, wb_ref[...], preferred_element_type=jnp.float32)


def _comb_mm2(aa, ab, ha, hb, ba, bb, wa, wb):
    return pl.pallas_call(
        _comb_body,
        grid=(GRID,),
        in_specs=[_row_spec, _row_spec, _row_spec, _row_spec,
                  _b_spec, _b_spec, _w_spec, _w_spec],
        out_specs=[_row_spec, _row_spec],
        out_shape=[_row_shape, _row_shape],
    )(aa, ab, ha, hb, ba, bb, wa, wb)


def _head_body(aa_ref, ab_ref, ha_ref, hb_ref, ba_ref, bb_ref,
               w1_ref, b1_ref, w2_ref, b2_ref, o_ref):
    t = (aa_ref[...] + ab_ref[...] + ha_ref[...] + hb_ref[...]
         + ba_ref[...] + bb_ref[...])
    h = _gelu(t)
    h = _gelu(jnp.dot(h, w1_ref[...], preferred_element_type=jnp.float32)
              + b1_ref[...])
    o_ref[...] = (jnp.dot(h, w2_ref[...], preferred_element_type=jnp.float32)
                  + b2_ref[...])


def _head(aa, ab, ha, hb, ba, bb, w1, b1, w2, b2):
    return pl.pallas_call(
        _head_body,
        grid=(GRID,),
        in_specs=[_row_spec, _row_spec, _row_spec, _row_spec,
                  _b_spec, _b_spec, _w_spec, _b_spec, _w_spec, _b_spec],
        out_specs=_row_spec,
        out_shape=_row_shape,
    )(aa, ab, ha, hb, ba, bb, w1, b1, w2, b2)


def kernel(x, edge_index_a, edge_index_b,
           W0a, b0a, W0b, b0b, W1a, b1a, W1b, b1b,
           Wh1, bh1, Wh2, bh2):
    ha, hb = _mm2(x, W0a, W0b)
    aa, ab = _seg2(ha, hb, edge_index_a, edge_index_b)
    h1a, h1b = _comb_mm2(aa, ab, ha, hb,
                         b0a.reshape(1, D), b0b.reshape(1, D), W1a, W1b)
    a1a, a1b = _seg2(h1a, h1b, edge_index_a, edge_index_b)
    out = _head(a1a, a1b, h1a, h1b,
                b1a.reshape(1, D), b1b.reshape(1, D),
                Wh1, bh1.reshape(1, D), Wh2, bh2.reshape(1, D))
    return out


# speedup vs baseline: 2.5637x; 1.7058x over previous
import functools

import jax
import jax.numpy as jnp
from jax import lax
from jax.experimental import pallas as pl
from jax.experimental.pallas import tpu as pltpu
from jax.experimental.pallas import tpu_sc as plsc

N = 10000
D = 128
E = 320000


NSUB = 16
CH = 128
NCHT = E // CH
WB = 640


@functools.cache
def _seg2_built():
    mesh = plsc.VectorSubcoreMesh(core_axis_name="c", subcore_axis_name="s")
    return functools.partial(
        pl.kernel,
        mesh=mesh,
        out_type=(
            jax.ShapeDtypeStruct((N, D), jnp.float32),
            jax.ShapeDtypeStruct((N, D), jnp.float32),
        ),
        scratch_types=[
            pltpu.VMEM((2, CH), jnp.int32),
            pltpu.VMEM((2, CH), jnp.int32),
            pltpu.VMEM((CH, D), jnp.float32),
            pltpu.VMEM((CH, D), jnp.float32),
            pltpu.VMEM_SHARED((N, D), jnp.float32),
            pltpu.SemaphoreType.DMA,
            pltpu.SemaphoreType.DMA,
            pltpu.SemaphoreType.DMA,
            pltpu.SemaphoreType.DMA,
        ],
    )(_seg2_body)


def _seg2(ha, hb, ea, eb):
    return _seg2_built()(ha, hb, ea, eb)


def _seg2_body(ha, hb, ea, eb, oa, ob, idxa, idxb, rowsa, rowsb, accum,
               gsema, gsemb, isema, isemb):
    idx = (idxa, idxb)
    rows = (rowsa, rowsb)
    gsem = (gsema, gsemb)
    isem = (isema, isemb)
    c = lax.axis_index("c")
    s = lax.axis_index("s")

    zv = jnp.zeros((16,), jnp.float32)

    def zrow(i, carry):
        for j in range(D // 16):
            rows[0][i, pl.ds(j * 16, 16)] = zv
        return carry

    lax.fori_loop(0, CH, zrow, 0)
    base_r = jnp.minimum(s * WB, N - WB)
    for k in range(WB // CH):
        pltpu.sync_copy(rows[0], accum.at[pl.ds(base_r + k * CH, CH)])
    plsc.subcore_barrier()

    cnt = jnp.where(s < NCHT % NSUB, NCHT // NSUB + 1, NCHT // NSUB)

    def run(h_ref, e_ref):
        def icopy(k, b):
            return pltpu.make_async_copy(
                e_ref.at[:, pl.ds((s + NSUB * k) * CH, CH)], idx[b], isem[b])

        def gcopy(b):
            return pltpu.make_async_copy(h_ref.at[idx[b].at[0]], rows[b],
                                         gsem[b])

        def sadd(b):
            pltpu.sync_copy(rows[b], accum.at[idx[b].at[1]], add=True)

        icopy(0, 0).start()
        icopy(1, 1).start()
        icopy(0, 0).wait()
        gcopy(0).start()
        icopy(1, 1).wait()
        gcopy(1).start()

        def pair(p, carry):
            k0 = 2 * p
            for b in range(2):
                gcopy(b).wait()
                sadd(b)
                icopy(k0 + 2 + b, b).start()
                icopy(k0 + 2 + b, b).wait()
                gcopy(b).start()
            return carry

        lax.fori_loop(0, cnt // 2 - 1, pair, 0)

        for b in range(2):
            gcopy(b).wait()
            sadd(b)

        @pl.when(cnt % 2 == 1)
        def _():
            k = cnt - 1
            icopy(k, 0).start()
            icopy(k, 0).wait()
            gcopy(0).start()
            gcopy(0).wait()
            sadd(0)

    @pl.when(c == 0)
    def _():
        run(ha, ea)

    @pl.when(c == 1)
    def _():
        run(hb, eb)

    plsc.subcore_barrier()

    @pl.when(c == 0)
    def _():
        pltpu.sync_copy(accum.at[pl.ds(base_r, WB)], oa.at[pl.ds(base_r, WB)])

    @pl.when(c == 1)
    def _():
        pltpu.sync_copy(accum.at[pl.ds(base_r, WB)], ob.at[pl.ds(base_r, WB)])


RB = 1000
GRID = N // RB

_row_spec = pl.BlockSpec((RB, D), lambda r: (r, 0))
_w_spec = pl.BlockSpec((D, D), lambda r: (0, 0))
_b_spec = pl.BlockSpec((1, D), lambda r: (0, 0))
_row_shape = jax.ShapeDtypeStruct((N, D), jnp.float32)

_INV_SQRT2 = 0.7071067811865476


def _gelu(t):
    return 0.5 * t * (1.0 + lax.erf(t * _INV_SQRT2))


def _mm2_body(x_ref, wa_ref, wb_ref, oa_ref, ob_ref):
    xb = x_ref[...]
    oa_ref[...] = jnp.dot(xb, wa_ref[...], preferred_element_type=jnp.float32)
    ob_ref[...] = jnp.dot(xb, wb_ref[...], preferred_element_type=jnp.float32)


def _mm2(x, wa, wb):
    return pl.pallas_call(
        _mm2_body,
        grid=(GRID,),
        in_specs=[_row_spec, _w_spec, _w_spec],
        out_specs=[_row_spec, _row_spec],
        out_shape=[_row_shape, _row_shape],
    )(x, wa, wb)


def _comb_body(aa_ref, ab_ref, ha_ref, hb_ref, ba_ref, bb_ref,
               wa_ref, wb_ref, oa_ref, ob_ref):
    t = (aa_ref[...] + ab_ref[...] + ha_ref[...] + hb_ref[...]
         + ba_ref[...] + bb_ref[...])
    h = _gelu(t)
    oa_ref[...] = jnp.dot(h, wa_ref[...], preferred_element_type=jnp.float32)
    ob_ref[...] = jnp.dot(h, wb_ref[...], preferred_element_type=jnp.float32)


def _comb_mm2(aa, ab, ha, hb, ba, bb, wa, wb):
    return pl.pallas_call(
        _comb_body,
        grid=(GRID,),
        in_specs=[_row_spec, _row_spec, _row_spec, _row_spec,
                  _b_spec, _b_spec, _w_spec, _w_spec],
        out_specs=[_row_spec, _row_spec],
        out_shape=[_row_shape, _row_shape],
    )(aa, ab, ha, hb, ba, bb, wa, wb)


def _head_body(aa_ref, ab_ref, ha_ref, hb_ref, ba_ref, bb_ref,
               w1_ref, b1_ref, w2_ref, b2_ref, o_ref):
    t = (aa_ref[...] + ab_ref[...] + ha_ref[...] + hb_ref[...]
         + ba_ref[...] + bb_ref[...])
    h = _gelu(t)
    h = _gelu(jnp.dot(h, w1_ref[...], preferred_element_type=jnp.float32)
              + b1_ref[...])
    o_ref[...] = (jnp.dot(h, w2_ref[...], preferred_element_type=jnp.float32)
                  + b2_ref[...])


def _head(aa, ab, ha, hb, ba, bb, w1, b1, w2, b2):
    return pl.pallas_call(
        _head_body,
        grid=(GRID,),
        in_specs=[_row_spec, _row_spec, _row_spec, _row_spec,
                  _b_spec, _b_spec, _w_spec, _b_spec, _w_spec, _b_spec],
        out_specs=_row_spec,
        out_shape=_row_shape,
    )(aa, ab, ha, hb, ba, bb, w1, b1, w2, b2)


def kernel(x, edge_index_a, edge_index_b,
           W0a, b0a, W0b, b0b, W1a, b1a, W1b, b1b,
           Wh1, bh1, Wh2, bh2):
    ha, hb = _mm2(x, W0a, W0b)
    aa, ab = _seg2(ha, hb, edge_index_a, edge_index_b)
    h1a, h1b = _comb_mm2(aa, ab, ha, hb,
                         b0a.reshape(1, D), b0b.reshape(1, D), W1a, W1b)
    a1a, a1b = _seg2(h1a, h1b, edge_index_a, edge_index_b)
    out = _head(a1a, a1b, h1a, h1b,
                b1a.reshape(1, D), b1b.reshape(1, D),
                Wh1, bh1.reshape(1, D), Wh2, bh2.reshape(1, D))
    return out


# speedup vs baseline: 2.9270x; 1.1417x over previous
import functools

import jax
import jax.numpy as jnp
from jax import lax
from jax.experimental import pallas as pl
from jax.experimental.pallas import tpu as pltpu
from jax.experimental.pallas import tpu_sc as plsc

N = 10000
D = 128
E = 320000


NSUB = 16
CH = 128
NCHT = E // CH
WB = 640


@functools.cache
def _seg2_built():
    mesh = plsc.VectorSubcoreMesh(core_axis_name="c", subcore_axis_name="s")
    return functools.partial(
        pl.kernel,
        mesh=mesh,
        out_type=(
            jax.ShapeDtypeStruct((N, D), jnp.float32),
            jax.ShapeDtypeStruct((N, D), jnp.float32),
        ),
        scratch_types=(
            [pltpu.VMEM((2, CH), jnp.int32) for _ in range(4)]
            + [pltpu.VMEM((CH, D), jnp.float32) for _ in range(2)]
            + [pltpu.VMEM_SHARED((N, D), jnp.float32)]
            + [pltpu.SemaphoreType.DMA for _ in range(6)]
        ),
    )(_seg2_body)


def _seg2(ha, hb, ea, eb):
    return _seg2_built()(ha, hb, ea, eb)


def _seg2_body(ha, hb, ea, eb, oa, ob, *scratch):
    idx = scratch[0:4]
    rows = scratch[4:6]
    accum = scratch[6]
    gsem = scratch[7:9]
    isem = scratch[9:13]
    c = lax.axis_index("c")
    s = lax.axis_index("s")

    zv = jnp.zeros((16,), jnp.float32)

    def zrow(i, carry):
        for j in range(D // 16):
            rows[0][i, pl.ds(j * 16, 16)] = zv
        return carry

    lax.fori_loop(0, CH, zrow, 0)
    base_r = jnp.minimum(s * WB, N - WB)
    for k in range(WB // CH):
        pltpu.sync_copy(rows[0], accum.at[pl.ds(base_r + k * CH, CH)])
    plsc.subcore_barrier()

    cnt = jnp.where(s < NCHT % NSUB, NCHT // NSUB + 1, NCHT // NSUB)

    def run(h_ref, e_ref):
        def icopy(k, q):
            return pltpu.make_async_copy(
                e_ref.at[:, pl.ds((s + NSUB * k) * CH, CH)], idx[q], isem[q])

        def gcopy(b, q):
            return pltpu.make_async_copy(h_ref.at[idx[q].at[0]], rows[b],
                                         gsem[b])

        def sadd(b, q):
            pltpu.sync_copy(rows[b], accum.at[idx[q].at[1]], add=True)

        for q in range(4):
            icopy(q, q).start()
        for b in range(2):
            icopy(b, b).wait()
            gcopy(b, b).start()

        def quad(p, carry):
            k0 = 4 * p
            for b in range(4):
                r = b % 2
                q2 = (b + 2) % 4
                gcopy(r, b).wait()
                sadd(r, b)
                icopy(k0 + 4 + b, b).start()
                icopy(k0 + 2 + b, q2).wait()
                gcopy(r, q2).start()
            return carry

        lax.fori_loop(0, cnt // 4 - 1, quad, 0)

        q0 = 4 * (cnt // 4) - 4
        for b in range(2):
            gcopy(b, b).wait()
            sadd(b, b)
            icopy(q0 + 2 + b, b + 2).wait()
            gcopy(b, b + 2).start()
        for b in range(2):
            gcopy(b, b + 2).wait()
            sadd(b, b + 2)

        def tail(k, carry):
            icopy(k, 0).start()
            icopy(k, 0).wait()
            gcopy(0, 0).start()
            gcopy(0, 0).wait()
            sadd(0, 0)
            return carry

        lax.fori_loop(4 * (cnt // 4), cnt, tail, 0)

    @pl.when(c == 0)
    def _():
        run(ha, ea)

    @pl.when(c == 1)
    def _():
        run(hb, eb)

    plsc.subcore_barrier()

    @pl.when(c == 0)
    def _():
        pltpu.sync_copy(accum.at[pl.ds(base_r, WB)], oa.at[pl.ds(base_r, WB)])

    @pl.when(c == 1)
    def _():
        pltpu.sync_copy(accum.at[pl.ds(base_r, WB)], ob.at[pl.ds(base_r, WB)])


RB = 1000
GRID = N // RB

_row_spec = pl.BlockSpec((RB, D), lambda r: (r, 0))
_w_spec = pl.BlockSpec((D, D), lambda r: (0, 0))
_b_spec = pl.BlockSpec((1, D), lambda r: (0, 0))
_row_shape = jax.ShapeDtypeStruct((N, D), jnp.float32)

_INV_SQRT2 = 0.7071067811865476


def _gelu(t):
    return 0.5 * t * (1.0 + lax.erf(t * _INV_SQRT2))


def _mm2_body(x_ref, wa_ref, wb_ref, oa_ref, ob_ref):
    xb = x_ref[...]
    oa_ref[...] = jnp.dot(xb, wa_ref[...], preferred_element_type=jnp.float32)
    ob_ref[...] = jnp.dot(xb, wb_ref[...], preferred_element_type=jnp.float32)


def _mm2(x, wa, wb):
    return pl.pallas_call(
        _mm2_body,
        grid=(GRID,),
        in_specs=[_row_spec, _w_spec, _w_spec],
        out_specs=[_row_spec, _row_spec],
        out_shape=[_row_shape, _row_shape],
    )(x, wa, wb)


def _comb_body(aa_ref, ab_ref, ha_ref, hb_ref, ba_ref, bb_ref,
               wa_ref, wb_ref, oa_ref, ob_ref):
    t = (aa_ref[...] + ab_ref[...] + ha_ref[...] + hb_ref[...]
         + ba_ref[...] + bb_ref[...])
    h = _gelu(t)
    oa_ref[...] = jnp.dot(h, wa_ref[...], preferred_element_type=jnp.float32)
    ob_ref[...] = jnp.dot(h, wb_ref[...], preferred_element_type=jnp.float32)


def _comb_mm2(aa, ab, ha, hb, ba, bb, wa, wb):
    return pl.pallas_call(
        _comb_body,
        grid=(GRID,),
        in_specs=[_row_spec, _row_spec, _row_spec, _row_spec,
                  _b_spec, _b_spec, _w_spec, _w_spec],
        out_specs=[_row_spec, _row_spec],
        out_shape=[_row_shape, _row_shape],
    )(aa, ab, ha, hb, ba, bb, wa, wb)


def _head_body(aa_ref, ab_ref, ha_ref, hb_ref, ba_ref, bb_ref,
               w1_ref, b1_ref, w2_ref, b2_ref, o_ref):
    t = (aa_ref[...] + ab_ref[...] + ha_ref[...] + hb_ref[...]
         + ba_ref[...] + bb_ref[...])
    h = _gelu(t)
    h = _gelu(jnp.dot(h, w1_ref[...], preferred_element_type=jnp.float32)
              + b1_ref[...])
    o_ref[...] = (jnp.dot(h, w2_ref[...], preferred_element_type=jnp.float32)
                  + b2_ref[...])


def _head(aa, ab, ha, hb, ba, bb, w1, b1, w2, b2):
    return pl.pallas_call(
        _head_body,
        grid=(GRID,),
        in_specs=[_row_spec, _row_spec, _row_spec, _row_spec,
                  _b_spec, _b_spec, _w_spec, _b_spec, _w_spec, _b_spec],
        out_specs=_row_spec,
        out_shape=_row_shape,
    )(aa, ab, ha, hb, ba, bb, w1, b1, w2, b2)


def kernel(x, edge_index_a, edge_index_b,
           W0a, b0a, W0b, b0b, W1a, b1a, W1b, b1b,
           Wh1, bh1, Wh2, bh2):
    ha, hb = _mm2(x, W0a, W0b)
    aa, ab = _seg2(ha, hb, edge_index_a, edge_index_b)
    h1a, h1b = _comb_mm2(aa, ab, ha, hb,
                         b0a.reshape(1, D), b0b.reshape(1, D), W1a, W1b)
    a1a, a1b = _seg2(h1a, h1b, edge_index_a, edge_index_b)
    out = _head(a1a, a1b, h1a, h1b,
                b1a.reshape(1, D), b1b.reshape(1, D),
                Wh1, bh1.reshape(1, D), Wh2, bh2.reshape(1, D))
    return out
